# Initial kernel scaffold; baseline (speedup 1.0000x reference)
import jax
import jax.numpy as jnp
from jax.experimental import pallas as pl


def kernel(x, edge_index, Wl1, Wr1, b1, Wl2, Wr2, b2, Wlin, blin, Wsc, bsc):
    raise NotImplementedError("write your pallas kernel here")


# speedup vs baseline: 7.5026x; 7.5026x over previous
import jax
import jax.numpy as jnp
from jax import lax
from jax.experimental import pallas as pl
from jax.experimental.pallas import tpu as pltpu
from jax.experimental.pallas import tpu_sc as plsc

N_NODES = 10000
N_EDGES = 320000
D = 128
D2 = 256

NC = 2
NS = 16
NW = NC * NS
E_PER_TILE = N_EDGES // NW
CHUNK = 125
N_CHUNKS = E_PER_TILE // CHUNK
IDX_BLK = 8
STRIPE = 632
STRIPE_LAST = N_NODES - 15 * STRIPE

_MESH = plsc.VectorSubcoreMesh(core_axis_name="c", subcore_axis_name="s")


def _zero_stripe(sid, zrows, acc_sh):
    @pl.when(sid < NS - 1)
    def _():
        pltpu.sync_copy(zrows.at[pl.ds(sid * STRIPE, STRIPE)],
                        acc_sh.at[pl.ds(sid * STRIPE, STRIPE)])

    @pl.when(sid == NS - 1)
    def _():
        pltpu.sync_copy(zrows.at[pl.ds(15 * STRIPE, STRIPE_LAST)],
                        acc_sh.at[pl.ds(15 * STRIPE, STRIPE_LAST)])


def _copyout_stripe(sid, cid, acc_sh, out_hbm):
    @pl.when(sid < NS - 1)
    def _():
        pltpu.sync_copy(acc_sh.at[pl.ds(sid * STRIPE, STRIPE)],
                        out_hbm.at[cid, pl.ds(sid * STRIPE, STRIPE)])

    @pl.when(sid == NS - 1)
    def _():
        pltpu.sync_copy(acc_sh.at[pl.ds(15 * STRIPE, STRIPE_LAST)],
                        out_hbm.at[cid, pl.ds(15 * STRIPE, STRIPE_LAST)])


def _sc_agg_body(y, srcs, dsts, zrows, agg_out, src_v, dst_v, rows_v,
                 agg_sh, sem):
    cid = lax.axis_index("c")
    sid = lax.axis_index("s")
    _zero_stripe(sid, zrows, agg_sh)
    plsc.subcore_barrier()

    def step_g(g, carry):
        pltpu.sync_copy(srcs.at[cid, sid, pl.ds(g * IDX_BLK, IDX_BLK)], src_v)
        pltpu.sync_copy(dsts.at[cid, sid, pl.ds(g * IDX_BLK, IDX_BLK)], dst_v)

        def step_j(j, carry2):
            pltpu.async_copy(y.at[src_v.at[j]], rows_v, sem).wait()
            pltpu.sync_copy(rows_v, agg_sh.at[dst_v.at[j]], add=True)
            return carry2

        return lax.fori_loop(0, IDX_BLK, step_j, carry)

    lax.fori_loop(0, N_CHUNKS // IDX_BLK, step_g, 0)

    plsc.subcore_barrier()
    _copyout_stripe(sid, cid, agg_sh, agg_out)


_sc_agg = pl.kernel(
    _sc_agg_body,
    mesh=_MESH,
    out_type=jax.ShapeDtypeStruct((NC, N_NODES, D), jnp.float32),
    scratch_types=[
        pltpu.VMEM((IDX_BLK, CHUNK), jnp.int32),
        pltpu.VMEM((IDX_BLK, CHUNK), jnp.int32),
        pltpu.VMEM((CHUNK, D), jnp.float32),
        pltpu.VMEM_SHARED((N_NODES, D), jnp.float32),
        pltpu.SemaphoreType.DMA,
    ],
)


def _sc_cnt_body(dsts, zrows, ones_h, cnt_out, dst_v, ones_v, cnt_sh, sem):
    cid = lax.axis_index("c")
    sid = lax.axis_index("s")
    pltpu.sync_copy(ones_h, ones_v)
    _zero_stripe(sid, zrows, cnt_sh)
    plsc.subcore_barrier()

    def step_g(g, carry):
        pltpu.sync_copy(dsts.at[cid, sid, pl.ds(g * IDX_BLK, IDX_BLK)], dst_v)

        def step_j(j, carry2):
            pltpu.sync_copy(ones_v, cnt_sh.at[dst_v.at[j]], add=True)
            return carry2

        return lax.fori_loop(0, IDX_BLK, step_j, carry)

    lax.fori_loop(0, N_CHUNKS // IDX_BLK, step_g, 0)

    plsc.subcore_barrier()
    _copyout_stripe(sid, cid, cnt_sh, cnt_out)


_sc_cnt = pl.kernel(
    _sc_cnt_body,
    mesh=_MESH,
    out_type=jax.ShapeDtypeStruct((NC, N_NODES, D), jnp.float32),
    scratch_types=[
        pltpu.VMEM((IDX_BLK, CHUNK), jnp.int32),
        pltpu.VMEM((CHUNK, D), jnp.float32),
        pltpu.VMEM_SHARED((N_NODES, D), jnp.float32),
        pltpu.SemaphoreType.DMA,
    ],
)


def _tc_layer1(agg, cnt, x, Wl, Wr, b):
    BLK = 2000

    def body(a_ref, c_ref, x_ref, wl_ref, wr_ref, b_ref, o_ref):
        c = jnp.maximum(c_ref[0, :, 0:1] + c_ref[1, :, 0:1], 1.0)
        mean = (a_ref[0] + a_ref[1]) / c
        o_ref[...] = jnp.maximum(
            jnp.dot(mean, wl_ref[...], preferred_element_type=jnp.float32)
            + jnp.dot(x_ref[...], wr_ref[...], preferred_element_type=jnp.float32)
            + b_ref[...],
            0.0,
        )

    return pl.pallas_call(
        body,
        grid=(N_NODES // BLK,),
        in_specs=[
            pl.BlockSpec((NC, BLK, D), lambda i: (0, i, 0)),
            pl.BlockSpec((NC, BLK, D), lambda i: (0, i, 0)),
            pl.BlockSpec((BLK, D), lambda i: (i, 0)),
            pl.BlockSpec((D, D), lambda i: (0, 0)),
            pl.BlockSpec((D, D), lambda i: (0, 0)),
            pl.BlockSpec((1, D), lambda i: (0, 0)),
        ],
        out_specs=pl.BlockSpec((BLK, D), lambda i: (i, 0)),
        out_shape=jax.ShapeDtypeStruct((N_NODES, D), jnp.float32),
    )(agg, cnt, x, Wl, Wr, b)


def _tc_final(agg2, cnt, h1, x, Wl2, Wr2, b2, Wlin, blin, Wsc, bsc):
    BLK = 2000

    def body(a_ref, c_ref, h1_ref, x_ref, wl2_ref, wr2_ref, b2_ref,
             wlin_ref, blin_ref, wsc_ref, bsc_ref, o_ref):
        c = jnp.maximum(c_ref[0, :, 0:1] + c_ref[1, :, 0:1], 1.0)
        mean = (a_ref[0] + a_ref[1]) / c
        h2 = jnp.maximum(
            jnp.dot(mean, wl2_ref[...], preferred_element_type=jnp.float32)
            + jnp.dot(h1_ref[...], wr2_ref[...], preferred_element_type=jnp.float32)
            + b2_ref[...],
            0.0,
        )
        o_ref[...] = (
            jnp.dot(h2, wlin_ref[...], preferred_element_type=jnp.float32)
            + blin_ref[...]
            + jnp.dot(x_ref[...], wsc_ref[...], preferred_element_type=jnp.float32)
            + bsc_ref[...]
        )

    return pl.pallas_call(
        body,
        grid=(N_NODES // BLK,),
        in_specs=[
            pl.BlockSpec((NC, BLK, D), lambda i: (0, i, 0)),
            pl.BlockSpec((NC, BLK, D), lambda i: (0, i, 0)),
            pl.BlockSpec((BLK, D), lambda i: (i, 0)),
            pl.BlockSpec((BLK, D), lambda i: (i, 0)),
            pl.BlockSpec((D, D2), lambda i: (0, 0)),
            pl.BlockSpec((D, D2), lambda i: (0, 0)),
            pl.BlockSpec((1, D2), lambda i: (0, 0)),
            pl.BlockSpec((D2, D), lambda i: (0, 0)),
            pl.BlockSpec((1, D), lambda i: (0, 0)),
            pl.BlockSpec((D, D), lambda i: (0, 0)),
            pl.BlockSpec((1, D), lambda i: (0, 0)),
        ],
        out_specs=pl.BlockSpec((BLK, D), lambda i: (i, 0)),
        out_shape=jax.ShapeDtypeStruct((N_NODES, D), jnp.float32),
    )(agg2, cnt, h1, x, Wl2, Wr2, b2, Wlin, blin, Wsc, bsc)


def kernel(x, edge_index, Wl1, Wr1, b1, Wl2, Wr2, b2, Wlin, blin, Wsc, bsc):
    ei = edge_index.astype(jnp.int32)
    srcs = ei[0].reshape(NC, NS, N_CHUNKS, CHUNK)
    dsts = ei[1].reshape(NC, NS, N_CHUNKS, CHUNK)
    del ei
    zrows = jnp.zeros((N_NODES, D), jnp.float32)
    ones_h = jnp.ones((CHUNK, D), jnp.float32)

    cnt = _sc_cnt(dsts, zrows, ones_h)
    agg1 = _sc_agg(x, srcs, dsts, zrows)
    h1 = _tc_layer1(agg1, cnt, x, Wl1, Wr1, b1.reshape(1, D))
    agg2 = _sc_agg(h1, srcs, dsts, zrows)
    out = _tc_final(agg2, cnt, h1, x, Wl2, Wr2, b2.reshape(1, D2),
                    Wlin, blin.reshape(1, D), Wsc, bsc.reshape(1, D))
    return out


# speedup vs baseline: 8.9425x; 1.1919x over previous
import jax
import jax.numpy as jnp
from jax import lax
from jax.experimental import pallas as pl
from jax.experimental.pallas import tpu as pltpu
from jax.experimental.pallas import tpu_sc as plsc

N_NODES = 10000
N_EDGES = 320000
D = 128
D2 = 256

NC = 2
NS = 16
NW = NC * NS
E_PER_TILE = N_EDGES // NW
CHUNK = 125
N_CHUNKS = E_PER_TILE // CHUNK
IDX_BLK = 8
STRIPE = 632
STRIPE_LAST = N_NODES - 15 * STRIPE

_MESH = plsc.VectorSubcoreMesh(core_axis_name="c", subcore_axis_name="s")


def _zero_stripe(sid, zrows, acc_sh):
    @pl.when(sid < NS - 1)
    def _():
        pltpu.sync_copy(zrows.at[pl.ds(sid * STRIPE, STRIPE)],
                        acc_sh.at[pl.ds(sid * STRIPE, STRIPE)])

    @pl.when(sid == NS - 1)
    def _():
        pltpu.sync_copy(zrows.at[pl.ds(15 * STRIPE, STRIPE_LAST)],
                        acc_sh.at[pl.ds(15 * STRIPE, STRIPE_LAST)])


def _copyout_stripe(sid, cid, acc_sh, out_hbm):
    @pl.when(sid < NS - 1)
    def _():
        pltpu.sync_copy(acc_sh.at[pl.ds(sid * STRIPE, STRIPE)],
                        out_hbm.at[cid, pl.ds(sid * STRIPE, STRIPE)])

    @pl.when(sid == NS - 1)
    def _():
        pltpu.sync_copy(acc_sh.at[pl.ds(15 * STRIPE, STRIPE_LAST)],
                        out_hbm.at[cid, pl.ds(15 * STRIPE, STRIPE_LAST)])


def _sc_agg_body(y, srcs, dsts, zrows, agg_out, src_v, dst_v, rows_v,
                 agg_sh, sem):
    cid = lax.axis_index("c")
    sid = lax.axis_index("s")
    _zero_stripe(sid, zrows, agg_sh)
    plsc.subcore_barrier()

    def step_g(g, carry):
        pltpu.sync_copy(srcs.at[cid, sid, pl.ds(g * IDX_BLK, IDX_BLK)], src_v)
        pltpu.sync_copy(dsts.at[cid, sid, pl.ds(g * IDX_BLK, IDX_BLK)], dst_v)
        pltpu.async_copy(y.at[src_v.at[0]], rows_v.at[0], sem)
        for j in range(IDX_BLK):
            pltpu.make_async_copy(y.at[src_v.at[j]], rows_v.at[j % 2],
                                  sem).wait()
            if j + 1 < IDX_BLK:
                pltpu.async_copy(y.at[src_v.at[j + 1]],
                                 rows_v.at[(j + 1) % 2], sem)
            pltpu.sync_copy(rows_v.at[j % 2], agg_sh.at[dst_v.at[j]],
                            add=True)
        return carry

    lax.fori_loop(0, N_CHUNKS // IDX_BLK, step_g, 0)

    plsc.subcore_barrier()
    _copyout_stripe(sid, cid, agg_sh, agg_out)


_sc_agg = pl.kernel(
    _sc_agg_body,
    mesh=_MESH,
    out_type=jax.ShapeDtypeStruct((NC, N_NODES, D), jnp.float32),
    scratch_types=[
        pltpu.VMEM((IDX_BLK, CHUNK), jnp.int32),
        pltpu.VMEM((IDX_BLK, CHUNK), jnp.int32),
        pltpu.VMEM((2, CHUNK, D), jnp.float32),
        pltpu.VMEM_SHARED((N_NODES, D), jnp.float32),
        pltpu.SemaphoreType.DMA,
    ],
)


def _sc_cnt_body(dsts, zrows, ones_h, cnt_out, dst_v, ones_v, cnt_sh, sem):
    cid = lax.axis_index("c")
    sid = lax.axis_index("s")
    pltpu.sync_copy(ones_h, ones_v)
    _zero_stripe(sid, zrows, cnt_sh)
    plsc.subcore_barrier()

    def step_g(g, carry):
        pltpu.sync_copy(dsts.at[cid, sid, pl.ds(g * IDX_BLK, IDX_BLK)], dst_v)

        def step_j(j, carry2):
            pltpu.sync_copy(ones_v, cnt_sh.at[dst_v.at[j]], add=True)
            return carry2

        return lax.fori_loop(0, IDX_BLK, step_j, carry)

    lax.fori_loop(0, N_CHUNKS // IDX_BLK, step_g, 0)

    plsc.subcore_barrier()
    _copyout_stripe(sid, cid, cnt_sh, cnt_out)


_sc_cnt = pl.kernel(
    _sc_cnt_body,
    mesh=_MESH,
    out_type=jax.ShapeDtypeStruct((NC, N_NODES, D), jnp.float32),
    scratch_types=[
        pltpu.VMEM((IDX_BLK, CHUNK), jnp.int32),
        pltpu.VMEM((CHUNK, D), jnp.float32),
        pltpu.VMEM_SHARED((N_NODES, D), jnp.float32),
        pltpu.SemaphoreType.DMA,
    ],
)


def _tc_layer1(agg, cnt, x, Wl, Wr, b):
    BLK = 2000

    def body(a_ref, c_ref, x_ref, wl_ref, wr_ref, b_ref, o_ref):
        c = jnp.maximum(c_ref[0, :, 0:1] + c_ref[1, :, 0:1], 1.0)
        mean = (a_ref[0] + a_ref[1]) / c
        o_ref[...] = jnp.maximum(
            jnp.dot(mean, wl_ref[...], preferred_element_type=jnp.float32)
            + jnp.dot(x_ref[...], wr_ref[...], preferred_element_type=jnp.float32)
            + b_ref[...],
            0.0,
        )

    return pl.pallas_call(
        body,
        grid=(N_NODES // BLK,),
        in_specs=[
            pl.BlockSpec((NC, BLK, D), lambda i: (0, i, 0)),
            pl.BlockSpec((NC, BLK, D), lambda i: (0, i, 0)),
            pl.BlockSpec((BLK, D), lambda i: (i, 0)),
            pl.BlockSpec((D, D), lambda i: (0, 0)),
            pl.BlockSpec((D, D), lambda i: (0, 0)),
            pl.BlockSpec((1, D), lambda i: (0, 0)),
        ],
        out_specs=pl.BlockSpec((BLK, D), lambda i: (i, 0)),
        out_shape=jax.ShapeDtypeStruct((N_NODES, D), jnp.float32),
    )(agg, cnt, x, Wl, Wr, b)


def _tc_final(agg2, cnt, h1, x, Wl2, Wr2, b2, Wlin, blin, Wsc, bsc):
    BLK = 2000

    def body(a_ref, c_ref, h1_ref, x_ref, wl2_ref, wr2_ref, b2_ref,
             wlin_ref, blin_ref, wsc_ref, bsc_ref, o_ref):
        c = jnp.maximum(c_ref[0, :, 0:1] + c_ref[1, :, 0:1], 1.0)
        mean = (a_ref[0] + a_ref[1]) / c
        h2 = jnp.maximum(
            jnp.dot(mean, wl2_ref[...], preferred_element_type=jnp.float32)
            + jnp.dot(h1_ref[...], wr2_ref[...], preferred_element_type=jnp.float32)
            + b2_ref[...],
            0.0,
        )
        o_ref[...] = (
            jnp.dot(h2, wlin_ref[...], preferred_element_type=jnp.float32)
            + blin_ref[...]
            + jnp.dot(x_ref[...], wsc_ref[...], preferred_element_type=jnp.float32)
            + bsc_ref[...]
        )

    return pl.pallas_call(
        body,
        grid=(N_NODES // BLK,),
        in_specs=[
            pl.BlockSpec((NC, BLK, D), lambda i: (0, i, 0)),
            pl.BlockSpec((NC, BLK, D), lambda i: (0, i, 0)),
            pl.BlockSpec((BLK, D), lambda i: (i, 0)),
            pl.BlockSpec((BLK, D), lambda i: (i, 0)),
            pl.BlockSpec((D, D2), lambda i: (0, 0)),
            pl.BlockSpec((D, D2), lambda i: (0, 0)),
            pl.BlockSpec((1, D2), lambda i: (0, 0)),
            pl.BlockSpec((D2, D), lambda i: (0, 0)),
            pl.BlockSpec((1, D), lambda i: (0, 0)),
            pl.BlockSpec((D, D), lambda i: (0, 0)),
            pl.BlockSpec((1, D), lambda i: (0, 0)),
        ],
        out_specs=pl.BlockSpec((BLK, D), lambda i: (i, 0)),
        out_shape=jax.ShapeDtypeStruct((N_NODES, D), jnp.float32),
    )(agg2, cnt, h1, x, Wl2, Wr2, b2, Wlin, blin, Wsc, bsc)


def kernel(x, edge_index, Wl1, Wr1, b1, Wl2, Wr2, b2, Wlin, blin, Wsc, bsc):
    ei = edge_index.astype(jnp.int32)
    srcs = ei[0].reshape(NC, NS, N_CHUNKS, CHUNK)
    dsts = ei[1].reshape(NC, NS, N_CHUNKS, CHUNK)
    del ei
    zrows = jnp.zeros((N_NODES, D), jnp.float32)
    ones_h = jnp.ones((CHUNK, D), jnp.float32)

    cnt = _sc_cnt(dsts, zrows, ones_h)
    agg1 = _sc_agg(x, srcs, dsts, zrows)
    h1 = _tc_layer1(agg1, cnt, x, Wl1, Wr1, b1.reshape(1, D))
    agg2 = _sc_agg(h1, srcs, dsts, zrows)
    out = _tc_final(agg2, cnt, h1, x, Wl2, Wr2, b2.reshape(1, D2),
                    Wlin, blin.reshape(1, D), Wsc, bsc.reshape(1, D))
    return out


# speedup vs baseline: 9.1265x; 1.0206x over previous
import jax
import jax.numpy as jnp
from jax import lax
from jax.experimental import pallas as pl
from jax.experimental.pallas import tpu as pltpu
from jax.experimental.pallas import tpu_sc as plsc

N_NODES = 10000
N_EDGES = 320000
D = 128
D2 = 256

NC = 2
NS = 16
NW = NC * NS
E_PER_TILE = N_EDGES // NW
CHUNK = 125
N_CHUNKS = E_PER_TILE // CHUNK
IDX_BLK = 16
N_BLKS = N_CHUNKS // IDX_BLK
STRIPE = 632
STRIPE_LAST = N_NODES - 15 * STRIPE

_MESH = plsc.VectorSubcoreMesh(core_axis_name="c", subcore_axis_name="s")


def _zero_stripe(sid, zrows, acc_sh):
    @pl.when(sid < NS - 1)
    def _():
        pltpu.sync_copy(zrows.at[pl.ds(sid * STRIPE, STRIPE)],
                        acc_sh.at[pl.ds(sid * STRIPE, STRIPE)])

    @pl.when(sid == NS - 1)
    def _():
        pltpu.sync_copy(zrows.at[pl.ds(15 * STRIPE, STRIPE_LAST)],
                        acc_sh.at[pl.ds(15 * STRIPE, STRIPE_LAST)])


def _copyout_stripe(sid, cid, acc_sh, out_hbm):
    @pl.when(sid < NS - 1)
    def _():
        pltpu.sync_copy(acc_sh.at[pl.ds(sid * STRIPE, STRIPE)],
                        out_hbm.at[cid, pl.ds(sid * STRIPE, STRIPE)])

    @pl.when(sid == NS - 1)
    def _():
        pltpu.sync_copy(acc_sh.at[pl.ds(15 * STRIPE, STRIPE_LAST)],
                        out_hbm.at[cid, pl.ds(15 * STRIPE, STRIPE_LAST)])


def _sc_agg_body(y, srcs, dsts, zrows, agg_out, src_v, dst_v, rows_v,
                 agg_sh, sem):
    cid = lax.axis_index("c")
    sid = lax.axis_index("s")
    _zero_stripe(sid, zrows, agg_sh)
    plsc.subcore_barrier()

    def step_g(g, carry):
        pltpu.sync_copy(srcs.at[cid, sid, pl.ds(g * IDX_BLK, IDX_BLK)], src_v)
        pltpu.sync_copy(dsts.at[cid, sid, pl.ds(g * IDX_BLK, IDX_BLK)], dst_v)
        pltpu.async_copy(y.at[src_v.at[0]], rows_v.at[0], sem)
        for j in range(IDX_BLK):
            pltpu.make_async_copy(y.at[src_v.at[j]], rows_v.at[j % 2],
                                  sem).wait()
            if j + 1 < IDX_BLK:
                pltpu.async_copy(y.at[src_v.at[j + 1]],
                                 rows_v.at[(j + 1) % 2], sem)
            pltpu.sync_copy(rows_v.at[j % 2], agg_sh.at[dst_v.at[j]],
                            add=True)
        return carry

    lax.fori_loop(0, N_BLKS, step_g, 0)

    plsc.subcore_barrier()
    _copyout_stripe(sid, cid, agg_sh, agg_out)


_sc_agg = pl.kernel(
    _sc_agg_body,
    mesh=_MESH,
    out_type=jax.ShapeDtypeStruct((NC, N_NODES, D), jnp.float32),
    scratch_types=[
        pltpu.VMEM((IDX_BLK, CHUNK), jnp.int32),
        pltpu.VMEM((IDX_BLK, CHUNK), jnp.int32),
        pltpu.VMEM((2, CHUNK, D), jnp.float32),
        pltpu.VMEM_SHARED((N_NODES, D), jnp.float32),
        pltpu.SemaphoreType.DMA,
    ],
)


def _sc_cnt_body(dsts, zrows, ones_h, cnt_out, dst_v, ones_v, cnt_sh, sem):
    cid = lax.axis_index("c")
    sid = lax.axis_index("s")
    pltpu.sync_copy(ones_h, ones_v)
    _zero_stripe(sid, zrows, cnt_sh)
    plsc.subcore_barrier()

    def step_g(g, carry):
        pltpu.sync_copy(dsts.at[cid, sid, pl.ds(g * IDX_BLK, IDX_BLK)], dst_v)

        def step_j(j, carry2):
            pltpu.sync_copy(ones_v, cnt_sh.at[dst_v.at[j]], add=True)
            return carry2

        return lax.fori_loop(0, IDX_BLK, step_j, carry)

    lax.fori_loop(0, N_BLKS, step_g, 0)

    plsc.subcore_barrier()
    _copyout_stripe(sid, cid, cnt_sh, cnt_out)


_sc_cnt = pl.kernel(
    _sc_cnt_body,
    mesh=_MESH,
    out_type=jax.ShapeDtypeStruct((NC, N_NODES, D), jnp.float32),
    scratch_types=[
        pltpu.VMEM((IDX_BLK, CHUNK), jnp.int32),
        pltpu.VMEM((CHUNK, D), jnp.float32),
        pltpu.VMEM_SHARED((N_NODES, D), jnp.float32),
        pltpu.SemaphoreType.DMA,
    ],
)

BLK = 2000


def _mm_spec(k, n):
    return pl.BlockSpec((k, n), lambda i: (0, 0))


def _row_spec(n):
    return pl.BlockSpec((BLK, n), lambda i: (i, 0))


def _tc_layer1a(agg, x, Wl, Wr, b):
    def body(a_ref, x_ref, wl_ref, wr_ref, b_ref, p_ref, r_ref):
        s = a_ref[0] + a_ref[1]
        p_ref[...] = jnp.dot(s, wl_ref[...],
                             preferred_element_type=jnp.float32)
        r_ref[...] = (
            jnp.dot(x_ref[...], wr_ref[...],
                    preferred_element_type=jnp.float32) + b_ref[...])

    return pl.pallas_call(
        body,
        grid=(N_NODES // BLK,),
        in_specs=[
            pl.BlockSpec((NC, BLK, D), lambda i: (0, i, 0)),
            _row_spec(D), _mm_spec(D, D), _mm_spec(D, D), _mm_spec(1, D),
        ],
        out_specs=[_row_spec(D), _row_spec(D)],
        out_shape=[jax.ShapeDtypeStruct((N_NODES, D), jnp.float32),
                   jax.ShapeDtypeStruct((N_NODES, D), jnp.float32)],
    )(agg, x, Wl, Wr, b)


def _tc_layer1b(P, R, cnt):
    def body(p_ref, r_ref, c_ref, o_ref):
        c = jnp.maximum(c_ref[0, :, 0:1] + c_ref[1, :, 0:1], 1.0)
        o_ref[...] = jnp.maximum(p_ref[...] / c + r_ref[...], 0.0)

    return pl.pallas_call(
        body,
        grid=(N_NODES // BLK,),
        in_specs=[
            _row_spec(D), _row_spec(D),
            pl.BlockSpec((NC, BLK, D), lambda i: (0, i, 0)),
        ],
        out_specs=_row_spec(D),
        out_shape=jax.ShapeDtypeStruct((N_NODES, D), jnp.float32),
    )(P, R, cnt)


def _tc_layer2a(h1, x, Wr2, b2, Wsc, bsc, blin):
    def body(h1_ref, x_ref, wr2_ref, b2_ref, wsc_ref, bsc_ref, blin_ref,
             bref, gref):
        bref[...] = (
            jnp.dot(h1_ref[...], wr2_ref[...],
                    preferred_element_type=jnp.float32) + b2_ref[...])
        gref[...] = (
            jnp.dot(x_ref[...], wsc_ref[...],
                    preferred_element_type=jnp.float32)
            + bsc_ref[...] + blin_ref[...])

    return pl.pallas_call(
        body,
        grid=(N_NODES // BLK,),
        in_specs=[
            _row_spec(D), _row_spec(D),
            _mm_spec(D, D2), _mm_spec(1, D2),
            _mm_spec(D, D), _mm_spec(1, D), _mm_spec(1, D),
        ],
        out_specs=[_row_spec(D2), _row_spec(D)],
        out_shape=[jax.ShapeDtypeStruct((N_NODES, D2), jnp.float32),
                   jax.ShapeDtypeStruct((N_NODES, D), jnp.float32)],
    )(h1, x, Wr2, b2, Wsc, bsc, blin)


def _tc_layer2b(agg2, cnt, B, G, Wl2, Wlin):
    def body(a_ref, c_ref, b_ref, g_ref, wl2_ref, wlin_ref, o_ref):
        c = jnp.maximum(c_ref[0, :, 0:1] + c_ref[1, :, 0:1], 1.0)
        s = a_ref[0] + a_ref[1]
        h2 = jnp.maximum(
            jnp.dot(s, wl2_ref[...],
                    preferred_element_type=jnp.float32) / c + b_ref[...],
            0.0,
        )
        o_ref[...] = (
            jnp.dot(h2, wlin_ref[...], preferred_element_type=jnp.float32)
            + g_ref[...])

    return pl.pallas_call(
        body,
        grid=(N_NODES // BLK,),
        in_specs=[
            pl.BlockSpec((NC, BLK, D), lambda i: (0, i, 0)),
            pl.BlockSpec((NC, BLK, D), lambda i: (0, i, 0)),
            _row_spec(D2), _row_spec(D),
            _mm_spec(D, D2), _mm_spec(D2, D),
        ],
        out_specs=_row_spec(D),
        out_shape=jax.ShapeDtypeStruct((N_NODES, D), jnp.float32),
    )(agg2, cnt, B, G, Wl2, Wlin)


def kernel(x, edge_index, Wl1, Wr1, b1, Wl2, Wr2, b2, Wlin, blin, Wsc, bsc):
    ei = edge_index.astype(jnp.int32)
    srcs = ei[0].reshape(NC, NS, N_CHUNKS, CHUNK)
    dsts = ei[1].reshape(NC, NS, N_CHUNKS, CHUNK)
    del ei
    zrows = jnp.zeros((N_NODES, D), jnp.float32)
    ones_h = jnp.ones((CHUNK, D), jnp.float32)

    agg1 = _sc_agg(x, srcs, dsts, zrows)
    cnt = _sc_cnt(dsts, zrows, ones_h)
    P1, R1 = _tc_layer1a(agg1, x, Wl1, Wr1, b1.reshape(1, D))
    h1 = _tc_layer1b(P1, R1, cnt)
    agg2 = _sc_agg(h1, srcs, dsts, zrows)
    B2, G2 = _tc_layer2a(h1, x, Wr2, b2.reshape(1, D2), Wsc,
                         bsc.reshape(1, D), blin.reshape(1, D))
    out = _tc_layer2b(agg2, cnt, B2, G2, Wl2, Wlin)
    return out


# speedup vs baseline: 9.2971x; 1.0187x over previous
import jax
import jax.numpy as jnp
from jax import lax
from jax.experimental import pallas as pl
from jax.experimental.pallas import tpu as pltpu
from jax.experimental.pallas import tpu_sc as plsc

N_NODES = 10000
N_EDGES = 320000
D = 128
D2 = 256

NC = 2
NS = 16
NW = NC * NS
E_PER_TILE = N_EDGES // NW
CHUNK = 125
N_CHUNKS = E_PER_TILE // CHUNK
IDX_BLK = 16
N_BLKS = N_CHUNKS // IDX_BLK
STRIPE = 632
STRIPE_LAST = N_NODES - 15 * STRIPE

_MESH = plsc.VectorSubcoreMesh(core_axis_name="c", subcore_axis_name="s")


def _zero_stripe(sid, zrows, acc_sh):
    @pl.when(sid < NS - 1)
    def _():
        pltpu.sync_copy(zrows.at[pl.ds(sid * STRIPE, STRIPE)],
                        acc_sh.at[pl.ds(sid * STRIPE, STRIPE)])

    @pl.when(sid == NS - 1)
    def _():
        pltpu.sync_copy(zrows.at[pl.ds(15 * STRIPE, STRIPE_LAST)],
                        acc_sh.at[pl.ds(15 * STRIPE, STRIPE_LAST)])


def _copyout_stripe(sid, cid, acc_sh, out_hbm):
    @pl.when(sid < NS - 1)
    def _():
        pltpu.sync_copy(acc_sh.at[pl.ds(sid * STRIPE, STRIPE)],
                        out_hbm.at[cid, pl.ds(sid * STRIPE, STRIPE)])

    @pl.when(sid == NS - 1)
    def _():
        pltpu.sync_copy(acc_sh.at[pl.ds(15 * STRIPE, STRIPE_LAST)],
                        out_hbm.at[cid, pl.ds(15 * STRIPE, STRIPE_LAST)])


def _sc_agg_body(y, srcs, dsts, zrows, agg_out, src_v, dst_v, rows_v,
                 agg_sh, sem):
    cid = lax.axis_index("c")
    sid = lax.axis_index("s")
    _zero_stripe(sid, zrows, agg_sh)
    plsc.subcore_barrier()

    def step_g(g, carry):
        pltpu.sync_copy(srcs.at[cid, sid, pl.ds(g * IDX_BLK, IDX_BLK)], src_v)
        pltpu.sync_copy(dsts.at[cid, sid, pl.ds(g * IDX_BLK, IDX_BLK)], dst_v)
        pltpu.async_copy(y.at[src_v.at[0]], rows_v.at[0], sem)
        for j in range(IDX_BLK):
            pltpu.make_async_copy(y.at[src_v.at[j]], rows_v.at[j % 2],
                                  sem).wait()
            if j + 1 < IDX_BLK:
                pltpu.async_copy(y.at[src_v.at[j + 1]],
                                 rows_v.at[(j + 1) % 2], sem)
            pltpu.sync_copy(rows_v.at[j % 2], agg_sh.at[dst_v.at[j]],
                            add=True)
        return carry

    lax.fori_loop(0, N_BLKS, step_g, 0)

    plsc.subcore_barrier()
    _copyout_stripe(sid, cid, agg_sh, agg_out)


_sc_agg = pl.kernel(
    _sc_agg_body,
    mesh=_MESH,
    out_type=jax.ShapeDtypeStruct((NC, N_NODES, D), jnp.float32),
    scratch_types=[
        pltpu.VMEM((IDX_BLK, CHUNK), jnp.int32),
        pltpu.VMEM((IDX_BLK, CHUNK), jnp.int32),
        pltpu.VMEM((2, CHUNK, D), jnp.float32),
        pltpu.VMEM_SHARED((N_NODES, D), jnp.float32),
        pltpu.SemaphoreType.DMA,
    ],
)


def _sc_cnt_body(dsts, zrows, ones_h, cnt_out, dst_v, ones_v, cnt_sh, sem):
    cid = lax.axis_index("c")
    sid = lax.axis_index("s")
    pltpu.sync_copy(ones_h, ones_v)
    _zero_stripe(sid, zrows, cnt_sh)
    plsc.subcore_barrier()

    def step_g(g, carry):
        pltpu.sync_copy(dsts.at[cid, sid, pl.ds(g * IDX_BLK, IDX_BLK)], dst_v)
        for j in range(IDX_BLK):
            pltpu.async_copy(ones_v, cnt_sh.at[dst_v.at[j]], sem, add=True)
        for j in range(IDX_BLK):
            pltpu.make_async_copy(ones_v, cnt_sh.at[dst_v.at[j]],
                                  sem).wait()
        return carry

    lax.fori_loop(0, N_BLKS, step_g, 0)

    plsc.subcore_barrier()
    _copyout_stripe(sid, cid, cnt_sh, cnt_out)


_sc_cnt = pl.kernel(
    _sc_cnt_body,
    mesh=_MESH,
    out_type=jax.ShapeDtypeStruct((NC, N_NODES, D), jnp.float32),
    scratch_types=[
        pltpu.VMEM((IDX_BLK, CHUNK), jnp.int32),
        pltpu.VMEM((CHUNK, D), jnp.float32),
        pltpu.VMEM_SHARED((N_NODES, D), jnp.float32),
        pltpu.SemaphoreType.DMA,
    ],
)

BLK = 2000


def _tc_layer1(agg, cnt, x, Wl, Wr, b):
    def body(a_ref, c_ref, x_ref, wl_ref, wr_ref, b_ref, o_ref):
        c = jnp.maximum(c_ref[0, :, 0:1] + c_ref[1, :, 0:1], 1.0)
        mean = (a_ref[0] + a_ref[1]) / c
        o_ref[...] = jnp.maximum(
            jnp.dot(mean, wl_ref[...], preferred_element_type=jnp.float32)
            + jnp.dot(x_ref[...], wr_ref[...], preferred_element_type=jnp.float32)
            + b_ref[...],
            0.0,
        )

    return pl.pallas_call(
        body,
        grid=(N_NODES // BLK,),
        in_specs=[
            pl.BlockSpec((NC, BLK, D), lambda i: (0, i, 0)),
            pl.BlockSpec((NC, BLK, D), lambda i: (0, i, 0)),
            pl.BlockSpec((BLK, D), lambda i: (i, 0)),
            pl.BlockSpec((D, D), lambda i: (0, 0)),
            pl.BlockSpec((D, D), lambda i: (0, 0)),
            pl.BlockSpec((1, D), lambda i: (0, 0)),
        ],
        out_specs=pl.BlockSpec((BLK, D), lambda i: (i, 0)),
        out_shape=jax.ShapeDtypeStruct((N_NODES, D), jnp.float32),
    )(agg, cnt, x, Wl, Wr, b)


def _tc_final(agg2, cnt, h1, x, Wl2, Wr2, b2, Wlin, blin, Wsc, bsc):
    def body(a_ref, c_ref, h1_ref, x_ref, wl2_ref, wr2_ref, b2_ref,
             wlin_ref, blin_ref, wsc_ref, bsc_ref, o_ref):
        c = jnp.maximum(c_ref[0, :, 0:1] + c_ref[1, :, 0:1], 1.0)
        mean = (a_ref[0] + a_ref[1]) / c
        h2 = jnp.maximum(
            jnp.dot(mean, wl2_ref[...], preferred_element_type=jnp.float32)
            + jnp.dot(h1_ref[...], wr2_ref[...], preferred_element_type=jnp.float32)
            + b2_ref[...],
            0.0,
        )
        o_ref[...] = (
            jnp.dot(h2, wlin_ref[...], preferred_element_type=jnp.float32)
            + blin_ref[...]
            + jnp.dot(x_ref[...], wsc_ref[...], preferred_element_type=jnp.float32)
            + bsc_ref[...]
        )

    return pl.pallas_call(
        body,
        grid=(N_NODES // BLK,),
        in_specs=[
            pl.BlockSpec((NC, BLK, D), lambda i: (0, i, 0)),
            pl.BlockSpec((NC, BLK, D), lambda i: (0, i, 0)),
            pl.BlockSpec((BLK, D), lambda i: (i, 0)),
            pl.BlockSpec((BLK, D), lambda i: (i, 0)),
            pl.BlockSpec((D, D2), lambda i: (0, 0)),
            pl.BlockSpec((D, D2), lambda i: (0, 0)),
            pl.BlockSpec((1, D2), lambda i: (0, 0)),
            pl.BlockSpec((D2, D), lambda i: (0, 0)),
            pl.BlockSpec((1, D), lambda i: (0, 0)),
            pl.BlockSpec((D, D), lambda i: (0, 0)),
            pl.BlockSpec((1, D), lambda i: (0, 0)),
        ],
        out_specs=pl.BlockSpec((BLK, D), lambda i: (i, 0)),
        out_shape=jax.ShapeDtypeStruct((N_NODES, D), jnp.float32),
    )(agg2, cnt, h1, x, Wl2, Wr2, b2, Wlin, blin, Wsc, bsc)


def kernel(x, edge_index, Wl1, Wr1, b1, Wl2, Wr2, b2, Wlin, blin, Wsc, bsc):
    ei = edge_index.astype(jnp.int32)
    srcs = ei[0].reshape(NC, NS, N_CHUNKS, CHUNK)
    dsts = ei[1].reshape(NC, NS, N_CHUNKS, CHUNK)
    del ei
    zrows = jnp.zeros((N_NODES, D), jnp.float32)
    ones_h = jnp.ones((CHUNK, D), jnp.float32)

    agg1 = _sc_agg(x, srcs, dsts, zrows)
    cnt = _sc_cnt(dsts, zrows, ones_h)
    h1 = _tc_layer1(agg1, cnt, x, Wl1, Wr1, b1.reshape(1, D))
    agg2 = _sc_agg(h1, srcs, dsts, zrows)
    out = _tc_final(agg2, cnt, h1, x, Wl2, Wr2, b2.reshape(1, D2),
                    Wlin, blin.reshape(1, D), Wsc, bsc.reshape(1, D))
    return out


# speedup vs baseline: 9.4147x; 1.0126x over previous
import jax
import jax.numpy as jnp
from jax import lax
from jax.experimental import pallas as pl
from jax.experimental.pallas import tpu as pltpu
from jax.experimental.pallas import tpu_sc as plsc

N_NODES = 10000
N_EDGES = 320000
D = 128
D2 = 256

NC = 2
NS = 16
NW = NC * NS
E_PER_TILE = N_EDGES // NW
CHUNK = 125
N_CHUNKS = E_PER_TILE // CHUNK
IDX_BLK = 16
N_BLKS = N_CHUNKS // IDX_BLK
STRIPE = 632
STRIPE_LAST = N_NODES - 15 * STRIPE

_MESH = plsc.VectorSubcoreMesh(core_axis_name="c", subcore_axis_name="s")


def _zero_stripe(sid, zrows, acc_sh):
    @pl.when(sid < NS - 1)
    def _():
        pltpu.sync_copy(zrows.at[pl.ds(sid * STRIPE, STRIPE)],
                        acc_sh.at[pl.ds(sid * STRIPE, STRIPE)])

    @pl.when(sid == NS - 1)
    def _():
        pltpu.sync_copy(zrows.at[pl.ds(15 * STRIPE, STRIPE_LAST)],
                        acc_sh.at[pl.ds(15 * STRIPE, STRIPE_LAST)])


def _copyout_stripe(sid, cid, acc_sh, out_hbm):
    @pl.when(sid < NS - 1)
    def _():
        pltpu.sync_copy(acc_sh.at[pl.ds(sid * STRIPE, STRIPE)],
                        out_hbm.at[cid, pl.ds(sid * STRIPE, STRIPE)])

    @pl.when(sid == NS - 1)
    def _():
        pltpu.sync_copy(acc_sh.at[pl.ds(15 * STRIPE, STRIPE_LAST)],
                        out_hbm.at[cid, pl.ds(15 * STRIPE, STRIPE_LAST)])


def _sc_body(with_cnt, y, srcs, dsts, zrows, *rest):
    if with_cnt:
        (ones_h, agg_out, cnt_out,
         src_v, dst_v, rows_v, acc_sh, sem) = rest
    else:
        agg_out, src_v, dst_v, rows_v, acc_sh, sem = rest
    cid = lax.axis_index("c")
    sid = lax.axis_index("s")
    pltpu.sync_copy(srcs.at[cid, sid, pl.ds(0, IDX_BLK)], src_v)
    pltpu.sync_copy(dsts.at[cid, sid, pl.ds(0, IDX_BLK)], dst_v)
    pltpu.async_copy(y.at[src_v.at[0]], rows_v.at[0], sem)
    _zero_stripe(sid, zrows, acc_sh)
    plsc.subcore_barrier()

    def step_g(g, carry):
        @pl.when(g > 0)
        def _():
            pltpu.sync_copy(srcs.at[cid, sid, pl.ds(g * IDX_BLK, IDX_BLK)],
                            src_v)
            pltpu.sync_copy(dsts.at[cid, sid, pl.ds(g * IDX_BLK, IDX_BLK)],
                            dst_v)
            pltpu.async_copy(y.at[src_v.at[0]], rows_v.at[0], sem)
        for j in range(IDX_BLK):
            pltpu.make_async_copy(y.at[src_v.at[j]], rows_v.at[j % 2],
                                  sem).wait()
            if j + 1 < IDX_BLK:
                pltpu.async_copy(y.at[src_v.at[j + 1]],
                                 rows_v.at[(j + 1) % 2], sem)
            pltpu.sync_copy(rows_v.at[j % 2], acc_sh.at[dst_v.at[j]],
                            add=True)
        return carry

    lax.fori_loop(0, N_BLKS, step_g, 0)

    plsc.subcore_barrier()
    _copyout_stripe(sid, cid, acc_sh, agg_out)

    if with_cnt:
        ones_v = rows_v.at[0]
        pltpu.sync_copy(ones_h, ones_v)
        plsc.subcore_barrier()
        _zero_stripe(sid, zrows, acc_sh)
        plsc.subcore_barrier()

        def cnt_g(g, carry):
            pltpu.sync_copy(dsts.at[cid, sid, pl.ds(g * IDX_BLK, IDX_BLK)],
                            dst_v)
            for j in range(IDX_BLK):
                pltpu.async_copy(ones_v, acc_sh.at[dst_v.at[j]], sem,
                                 add=True)
            for j in range(IDX_BLK):
                pltpu.make_async_copy(ones_v, acc_sh.at[dst_v.at[j]],
                                      sem).wait()
            return carry

        lax.fori_loop(0, N_BLKS, cnt_g, 0)
        plsc.subcore_barrier()
        _copyout_stripe(sid, cid, acc_sh, cnt_out)


import functools as _ft

_sc_agg_cnt = pl.kernel(
    _ft.partial(_sc_body, True),
    mesh=_MESH,
    out_type=[
        jax.ShapeDtypeStruct((NC, N_NODES, D), jnp.float32),
        jax.ShapeDtypeStruct((NC, N_NODES, D), jnp.float32),
    ],
    scratch_types=[
        pltpu.VMEM((IDX_BLK, CHUNK), jnp.int32),
        pltpu.VMEM((IDX_BLK, CHUNK), jnp.int32),
        pltpu.VMEM((2, CHUNK, D), jnp.float32),
        pltpu.VMEM_SHARED((N_NODES, D), jnp.float32),
        pltpu.SemaphoreType.DMA,
    ],
)

_sc_agg = pl.kernel(
    _ft.partial(_sc_body, False),
    mesh=_MESH,
    out_type=jax.ShapeDtypeStruct((NC, N_NODES, D), jnp.float32),
    scratch_types=[
        pltpu.VMEM((IDX_BLK, CHUNK), jnp.int32),
        pltpu.VMEM((IDX_BLK, CHUNK), jnp.int32),
        pltpu.VMEM((2, CHUNK, D), jnp.float32),
        pltpu.VMEM_SHARED((N_NODES, D), jnp.float32),
        pltpu.SemaphoreType.DMA,
    ],
)

BLK = 2000


def _tc_layer1(agg, cnt, x, Wl, Wr, b):
    def body(a_ref, c_ref, x_ref, wl_ref, wr_ref, b_ref, o_ref):
        c = jnp.maximum(c_ref[0, :, 0:1] + c_ref[1, :, 0:1], 1.0)
        mean = (a_ref[0] + a_ref[1]) / c
        o_ref[...] = jnp.maximum(
            jnp.dot(mean, wl_ref[...], preferred_element_type=jnp.float32)
            + jnp.dot(x_ref[...], wr_ref[...], preferred_element_type=jnp.float32)
            + b_ref[...],
            0.0,
        )

    return pl.pallas_call(
        body,
        grid=(N_NODES // BLK,),
        in_specs=[
            pl.BlockSpec((NC, BLK, D), lambda i: (0, i, 0)),
            pl.BlockSpec((NC, BLK, D), lambda i: (0, i, 0)),
            pl.BlockSpec((BLK, D), lambda i: (i, 0)),
            pl.BlockSpec((D, D), lambda i: (0, 0)),
            pl.BlockSpec((D, D), lambda i: (0, 0)),
            pl.BlockSpec((1, D), lambda i: (0, 0)),
        ],
        out_specs=pl.BlockSpec((BLK, D), lambda i: (i, 0)),
        out_shape=jax.ShapeDtypeStruct((N_NODES, D), jnp.float32),
    )(agg, cnt, x, Wl, Wr, b)


def _tc_final(agg2, cnt, h1, x, Wl2, Wr2, b2, Wlin, blin, Wsc, bsc):
    def body(a_ref, c_ref, h1_ref, x_ref, wl2_ref, wr2_ref, b2_ref,
             wlin_ref, blin_ref, wsc_ref, bsc_ref, o_ref):
        c = jnp.maximum(c_ref[0, :, 0:1] + c_ref[1, :, 0:1], 1.0)
        mean = (a_ref[0] + a_ref[1]) / c
        h2 = jnp.maximum(
            jnp.dot(mean, wl2_ref[...], preferred_element_type=jnp.float32)
            + jnp.dot(h1_ref[...], wr2_ref[...], preferred_element_type=jnp.float32)
            + b2_ref[...],
            0.0,
        )
        o_ref[...] = (
            jnp.dot(h2, wlin_ref[...], preferred_element_type=jnp.float32)
            + blin_ref[...]
            + jnp.dot(x_ref[...], wsc_ref[...], preferred_element_type=jnp.float32)
            + bsc_ref[...]
        )

    return pl.pallas_call(
        body,
        grid=(N_NODES // BLK,),
        in_specs=[
            pl.BlockSpec((NC, BLK, D), lambda i: (0, i, 0)),
            pl.BlockSpec((NC, BLK, D), lambda i: (0, i, 0)),
            pl.BlockSpec((BLK, D), lambda i: (i, 0)),
            pl.BlockSpec((BLK, D), lambda i: (i, 0)),
            pl.BlockSpec((D, D2), lambda i: (0, 0)),
            pl.BlockSpec((D, D2), lambda i: (0, 0)),
            pl.BlockSpec((1, D2), lambda i: (0, 0)),
            pl.BlockSpec((D2, D), lambda i: (0, 0)),
            pl.BlockSpec((1, D), lambda i: (0, 0)),
            pl.BlockSpec((D, D), lambda i: (0, 0)),
            pl.BlockSpec((1, D), lambda i: (0, 0)),
        ],
        out_specs=pl.BlockSpec((BLK, D), lambda i: (i, 0)),
        out_shape=jax.ShapeDtypeStruct((N_NODES, D), jnp.float32),
    )(agg2, cnt, h1, x, Wl2, Wr2, b2, Wlin, blin, Wsc, bsc)


def kernel(x, edge_index, Wl1, Wr1, b1, Wl2, Wr2, b2, Wlin, blin, Wsc, bsc):
    ei = edge_index.astype(jnp.int32)
    srcs = ei[0].reshape(NC, NS, N_CHUNKS, CHUNK)
    dsts = ei[1].reshape(NC, NS, N_CHUNKS, CHUNK)
    del ei
    zrows = jnp.zeros((N_NODES, D), jnp.float32)
    ones_h = jnp.ones((CHUNK, D), jnp.float32)

    agg1, cnt = _sc_agg_cnt(x, srcs, dsts, zrows, ones_h)
    h1 = _tc_layer1(agg1, cnt, x, Wl1, Wr1, b1.reshape(1, D))
    agg2 = _sc_agg(h1, srcs, dsts, zrows)
    out = _tc_final(agg2, cnt, h1, x, Wl2, Wr2, b2.reshape(1, D2),
                    Wlin, blin.reshape(1, D), Wsc, bsc.reshape(1, D))
    return out


# speedup vs baseline: 9.4961x; 1.0086x over previous
import jax
import jax.numpy as jnp
from jax import lax
from jax.experimental import pallas as pl
from jax.experimental.pallas import tpu as pltpu
from jax.experimental.pallas import tpu_sc as plsc

N_NODES = 10000
N_EDGES = 320000
D = 128
D2 = 256

NC = 2
NS = 16
NW = NC * NS
E_PER_TILE = N_EDGES // NW
CHUNK = 125
N_CHUNKS = E_PER_TILE // CHUNK
IDX_BLK = 16
N_BLKS = N_CHUNKS // IDX_BLK
STRIPE = 632
STRIPE_LAST = N_NODES - 15 * STRIPE

_MESH = plsc.VectorSubcoreMesh(core_axis_name="c", subcore_axis_name="s")


def _zero_stripe(sid, zrows, acc_sh):
    @pl.when(sid < NS - 1)
    def _():
        pltpu.sync_copy(zrows.at[pl.ds(sid * STRIPE, STRIPE)],
                        acc_sh.at[pl.ds(sid * STRIPE, STRIPE)])

    @pl.when(sid == NS - 1)
    def _():
        pltpu.sync_copy(zrows.at[pl.ds(15 * STRIPE, STRIPE_LAST)],
                        acc_sh.at[pl.ds(15 * STRIPE, STRIPE_LAST)])


def _copyout_stripe(sid, cid, acc_sh, out_hbm):
    @pl.when(sid < NS - 1)
    def _():
        pltpu.sync_copy(acc_sh.at[pl.ds(sid * STRIPE, STRIPE)],
                        out_hbm.at[cid, pl.ds(sid * STRIPE, STRIPE)])

    @pl.when(sid == NS - 1)
    def _():
        pltpu.sync_copy(acc_sh.at[pl.ds(15 * STRIPE, STRIPE_LAST)],
                        out_hbm.at[cid, pl.ds(15 * STRIPE, STRIPE_LAST)])


def _sc_body(with_cnt, y, srcs, dsts, zrows, *rest):
    if with_cnt:
        (ones_h, agg_out, cnt_out,
         src_v, dst_v, rows_v, acc_sh, sem) = rest
    else:
        agg_out, src_v, dst_v, rows_v, acc_sh, sem = rest
    cid = lax.axis_index("c")
    sid = lax.axis_index("s")
    pltpu.sync_copy(srcs.at[cid, sid, pl.ds(0, IDX_BLK)], src_v)
    pltpu.sync_copy(dsts.at[cid, sid, pl.ds(0, IDX_BLK)], dst_v)
    pltpu.async_copy(y.at[src_v.at[0]], rows_v.at[0], sem)
    _zero_stripe(sid, zrows, acc_sh)
    plsc.subcore_barrier()

    def step_g(g, carry):
        @pl.when(g > 0)
        def _():
            pltpu.sync_copy(srcs.at[cid, sid, pl.ds(g * IDX_BLK, IDX_BLK)],
                            src_v)
            pltpu.sync_copy(dsts.at[cid, sid, pl.ds(g * IDX_BLK, IDX_BLK)],
                            dst_v)
            pltpu.async_copy(y.at[src_v.at[0]], rows_v.at[0], sem)
        for j in range(IDX_BLK):
            pltpu.make_async_copy(y.at[src_v.at[j]], rows_v.at[j % 2],
                                  sem).wait()
            if j + 1 < IDX_BLK:
                pltpu.async_copy(y.at[src_v.at[j + 1]],
                                 rows_v.at[(j + 1) % 2], sem)
            pltpu.sync_copy(rows_v.at[j % 2], acc_sh.at[dst_v.at[j]],
                            add=True)
        return carry

    lax.fori_loop(0, N_BLKS, step_g, 0)

    plsc.subcore_barrier()
    _copyout_stripe(sid, cid, acc_sh, agg_out)

    if with_cnt:
        ones_v = rows_v.at[0]
        pltpu.sync_copy(ones_h, ones_v)
        _zero_stripe(sid, zrows, acc_sh)
        plsc.subcore_barrier()

        def cnt_g(g, carry):
            pltpu.sync_copy(dsts.at[cid, sid, pl.ds(g * IDX_BLK, IDX_BLK)],
                            dst_v)
            for j in range(IDX_BLK):
                pltpu.async_copy(ones_v, acc_sh.at[dst_v.at[j]], sem,
                                 add=True)
            for j in range(IDX_BLK):
                pltpu.make_async_copy(ones_v, acc_sh.at[dst_v.at[j]],
                                      sem).wait()
            return carry

        lax.fori_loop(0, N_BLKS, cnt_g, 0)
        plsc.subcore_barrier()
        _copyout_stripe(sid, cid, acc_sh, cnt_out)


import functools as _ft

_sc_agg_cnt = pl.kernel(
    _ft.partial(_sc_body, True),
    mesh=_MESH,
    out_type=[
        jax.ShapeDtypeStruct((NC, N_NODES, D), jnp.float32),
        jax.ShapeDtypeStruct((NC, N_NODES, D), jnp.float32),
    ],
    scratch_types=[
        pltpu.VMEM((IDX_BLK, CHUNK), jnp.int32),
        pltpu.VMEM((IDX_BLK, CHUNK), jnp.int32),
        pltpu.VMEM((2, CHUNK, D), jnp.float32),
        pltpu.VMEM_SHARED((N_NODES, D), jnp.float32),
        pltpu.SemaphoreType.DMA,
    ],
)

_sc_agg = pl.kernel(
    _ft.partial(_sc_body, False),
    mesh=_MESH,
    out_type=jax.ShapeDtypeStruct((NC, N_NODES, D), jnp.float32),
    scratch_types=[
        pltpu.VMEM((IDX_BLK, CHUNK), jnp.int32),
        pltpu.VMEM((IDX_BLK, CHUNK), jnp.int32),
        pltpu.VMEM((2, CHUNK, D), jnp.float32),
        pltpu.VMEM_SHARED((N_NODES, D), jnp.float32),
        pltpu.SemaphoreType.DMA,
    ],
)

BLK = 2000


def _tc_layer1(agg, cnt, x, Wl, Wr, b):
    def body(a_ref, c_ref, x_ref, wl_ref, wr_ref, b_ref, o_ref):
        c = jnp.maximum(c_ref[0, :, 0:1] + c_ref[1, :, 0:1], 1.0)
        mean = (a_ref[0] + a_ref[1]) / c
        o_ref[...] = jnp.maximum(
            jnp.dot(mean, wl_ref[...], preferred_element_type=jnp.float32)
            + jnp.dot(x_ref[...], wr_ref[...], preferred_element_type=jnp.float32)
            + b_ref[...],
            0.0,
        )

    return pl.pallas_call(
        body,
        grid=(N_NODES // BLK,),
        in_specs=[
            pl.BlockSpec((NC, BLK, D), lambda i: (0, i, 0)),
            pl.BlockSpec((NC, BLK, D), lambda i: (0, i, 0)),
            pl.BlockSpec((BLK, D), lambda i: (i, 0)),
            pl.BlockSpec((D, D), lambda i: (0, 0)),
            pl.BlockSpec((D, D), lambda i: (0, 0)),
            pl.BlockSpec((1, D), lambda i: (0, 0)),
        ],
        out_specs=pl.BlockSpec((BLK, D), lambda i: (i, 0)),
        out_shape=jax.ShapeDtypeStruct((N_NODES, D), jnp.float32),
    )(agg, cnt, x, Wl, Wr, b)


def _tc_final(agg2, cnt, h1, x, Wl2, Wr2, b2, Wlin, blin, Wsc, bsc):
    def body(a_ref, c_ref, h1_ref, x_ref, wl2_ref, wr2_ref, b2_ref,
             wlin_ref, blin_ref, wsc_ref, bsc_ref, o_ref):
        c = jnp.maximum(c_ref[0, :, 0:1] + c_ref[1, :, 0:1], 1.0)
        mean = (a_ref[0] + a_ref[1]) / c
        h2 = jnp.maximum(
            jnp.dot(mean, wl2_ref[...], preferred_element_type=jnp.float32)
            + jnp.dot(h1_ref[...], wr2_ref[...], preferred_element_type=jnp.float32)
            + b2_ref[...],
            0.0,
        )
        o_ref[...] = (
            jnp.dot(h2, wlin_ref[...], preferred_element_type=jnp.float32)
            + blin_ref[...]
            + jnp.dot(x_ref[...], wsc_ref[...], preferred_element_type=jnp.float32)
            + bsc_ref[...]
        )

    return pl.pallas_call(
        body,
        grid=(N_NODES // BLK,),
        in_specs=[
            pl.BlockSpec((NC, BLK, D), lambda i: (0, i, 0)),
            pl.BlockSpec((NC, BLK, D), lambda i: (0, i, 0)),
            pl.BlockSpec((BLK, D), lambda i: (i, 0)),
            pl.BlockSpec((BLK, D), lambda i: (i, 0)),
            pl.BlockSpec((D, D2), lambda i: (0, 0)),
            pl.BlockSpec((D, D2), lambda i: (0, 0)),
            pl.BlockSpec((1, D2), lambda i: (0, 0)),
            pl.BlockSpec((D2, D), lambda i: (0, 0)),
            pl.BlockSpec((1, D), lambda i: (0, 0)),
            pl.BlockSpec((D, D), lambda i: (0, 0)),
            pl.BlockSpec((1, D), lambda i: (0, 0)),
        ],
        out_specs=pl.BlockSpec((BLK, D), lambda i: (i, 0)),
        out_shape=jax.ShapeDtypeStruct((N_NODES, D), jnp.float32),
    )(agg2, cnt, h1, x, Wl2, Wr2, b2, Wlin, blin, Wsc, bsc)


def kernel(x, edge_index, Wl1, Wr1, b1, Wl2, Wr2, b2, Wlin, blin, Wsc, bsc):
    ei = edge_index.astype(jnp.int32)
    srcs = ei[0].reshape(NC, NS, N_CHUNKS, CHUNK)
    dsts = ei[1].reshape(NC, NS, N_CHUNKS, CHUNK)
    del ei
    zrows = jnp.zeros((N_NODES, D), jnp.float32)
    ones_h = jnp.ones((CHUNK, D), jnp.float32)

    agg1, cnt = _sc_agg_cnt(x, srcs, dsts, zrows, ones_h)
    h1 = _tc_layer1(agg1, cnt, x, Wl1, Wr1, b1.reshape(1, D))
    agg2 = _sc_agg(h1, srcs, dsts, zrows)
    out = _tc_final(agg2, cnt, h1, x, Wl2, Wr2, b2.reshape(1, D2),
                    Wlin, blin.reshape(1, D), Wsc, bsc.reshape(1, D))
    return out
